# baseline (speedup 1.0000x reference)
import functools

import jax
import jax.numpy as jnp
from jax import lax
from jax.experimental import pallas as pl
from jax.experimental.pallas import tpu as pltpu
from jax.experimental.pallas import tpu_sc as plsc

NC = 2
NS = 16
NW = NC * NS
K = 80
FW = 128


def _sc_mesh():
    return plsc.VectorSubcoreMesh(
        core_axis_name="c", subcore_axis_name="s",
        num_cores=NC, num_subcores=NS)


def _deg_partials(dst_t, zslab, ones_k, n_pad, slab):
    j_steps = dst_t.shape[1]

    @functools.partial(
        pl.kernel,
        out_type=jax.ShapeDtypeStruct((NC * n_pad,), jnp.float32),
        mesh=_sc_mesh(),
        scratch_types=[
            pltpu.VMEM((j_steps, K), jnp.int32),
            pltpu.VMEM((K,), jnp.float32),
            pltpu.VMEM_SHARED((n_pad,), jnp.float32),
            pltpu.SemaphoreType.DMA,
        ],
    )
    def k(dst_hbm, z_hbm, ones_hbm, out_hbm, idx_v, ones_v, acc, sem):
        cid = lax.axis_index("c")
        sid = lax.axis_index("s")
        gid = cid * NS + sid
        pltpu.sync_copy(z_hbm, acc.at[pl.ds(sid * slab, slab)])
        pltpu.sync_copy(ones_hbm, ones_v)
        pltpu.sync_copy(dst_hbm.at[gid], idx_v)
        plsc.subcore_barrier()

        def body(j, carry):
            pltpu.sync_copy(ones_v, acc.at[idx_v.at[j]], add=True)
            return carry

        lax.fori_loop(0, j_steps, body, 0)
        plsc.subcore_barrier()
        pltpu.sync_copy(acc.at[pl.ds(sid * slab, slab)],
                        out_hbm.at[pl.ds(cid * n_pad + sid * slab, slab)])

    return k(dst_t, zslab, ones_k)


def _segsum_partials(hp, src_t, dst_t, zslab, n_pad, slab):
    f = hp.shape[1]
    kk = {16: 1008, 32: 720, 64: 240, 128: 48}[f]
    src_t = src_t.reshape(NW, -1, kk)
    dst_t = dst_t.reshape(NW, -1, kk)
    j_steps = src_t.shape[1]

    assert j_steps % 2 == 0

    @functools.partial(
        pl.kernel,
        out_type=jax.ShapeDtypeStruct((NC, n_pad, f), jnp.float32),
        mesh=_sc_mesh(),
        scratch_types=[
            pltpu.VMEM_SHARED((n_pad, f), jnp.float32),
            pltpu.VMEM((j_steps, kk), jnp.int32),
            pltpu.VMEM((j_steps, kk), jnp.int32),
            pltpu.VMEM((kk, f), jnp.float32),
            pltpu.VMEM((kk, f), jnp.float32),
            pltpu.SemaphoreType.DMA,
            pltpu.SemaphoreType.DMA,
            pltpu.SemaphoreType.DMA,
            pltpu.SemaphoreType.DMA,
        ],
        compiler_params=pltpu.CompilerParams(use_tc_tiling_on_sc=False),
    )
    def k(hp_hbm, src_hbm, dst_hbm, z_hbm, out_hbm,
          acc, src_v, dst_v, rows_a, rows_b, gsem_a, gsem_b, ssem_a, ssem_b):
        cid = lax.axis_index("c")
        sid = lax.axis_index("s")
        gid = cid * NS + sid
        pltpu.sync_copy(z_hbm, acc.at[pl.ds(sid * slab, slab)])
        pltpu.sync_copy(src_hbm.at[gid], src_v)
        pltpu.sync_copy(dst_hbm.at[gid], dst_v)
        plsc.subcore_barrier()

        def g_start(j, buf, sem):
            pltpu.async_copy(hp_hbm.at[src_v.at[j]], buf, sem)

        def g_wait(j, buf, sem):
            pltpu.make_async_copy(hp_hbm.at[src_v.at[j]], buf, sem).wait()

        def s_start(j, buf, sem):
            pltpu.async_copy(buf, acc.at[dst_v.at[j]], sem, add=True)

        def s_wait(j, buf, sem):
            pltpu.make_async_copy(buf, acc.at[dst_v.at[j]], sem).wait()

        g_start(0, rows_a, gsem_a)

        def body(jj, carry):
            j0 = 2 * jj
            j1 = j0 + 1
            g_wait(j0, rows_a, gsem_a)
            g_start(j1, rows_b, gsem_b)
            s_start(j0, rows_a, ssem_a)
            s_wait(j0, rows_a, ssem_a)

            @pl.when(jj != j_steps // 2 - 1)
            def _():
                g_start(j0 + 2, rows_a, gsem_a)

            g_wait(j1, rows_b, gsem_b)
            s_start(j1, rows_b, ssem_b)
            s_wait(j1, rows_b, ssem_b)
            return carry

        lax.fori_loop(0, j_steps // 2, body, 0)
        plsc.subcore_barrier()
        pltpu.sync_copy(acc.at[pl.ds(sid * slab, slab)],
                        out_hbm.at[cid, pl.ds(sid * slab, slab)])

    return k(hp, src_t, dst_t, zslab)


def _segsum_featsplit(hp2, src2, dst, zslab, n_pad, slab):
    fh = hp2.shape[1]
    kk = 240
    src_t = src2.reshape(NC * NS, -1, kk)
    dst_t = dst.reshape(NS, -1, kk)
    j_steps = src_t.shape[1]
    assert j_steps % 2 == 0 and j_steps == dst_t.shape[1]

    @functools.partial(
        pl.kernel,
        out_type=jax.ShapeDtypeStruct((NC, n_pad, fh), jnp.float32),
        mesh=_sc_mesh(),
        scratch_types=[
            pltpu.VMEM_SHARED((n_pad, fh), jnp.float32),
            pltpu.VMEM((j_steps, kk), jnp.int32),
            pltpu.VMEM((j_steps, kk), jnp.int32),
            pltpu.VMEM((kk, fh), jnp.float32),
            pltpu.VMEM((kk, fh), jnp.float32),
            pltpu.SemaphoreType.DMA,
            pltpu.SemaphoreType.DMA,
            pltpu.SemaphoreType.DMA,
            pltpu.SemaphoreType.DMA,
        ],
        compiler_params=pltpu.CompilerParams(use_tc_tiling_on_sc=False),
    )
    def k(hp_hbm, src_hbm, dst_hbm, z_hbm, out_hbm,
          acc, src_v, dst_v, rows_a, rows_b,
          gsem_a, gsem_b, ssem_a, ssem_b):
        cid = lax.axis_index("c")
        sid = lax.axis_index("s")
        pltpu.sync_copy(z_hbm, acc.at[pl.ds(sid * slab, slab)])
        pltpu.sync_copy(src_hbm.at[cid * NS + sid], src_v)
        pltpu.sync_copy(dst_hbm.at[sid], dst_v)
        plsc.subcore_barrier()

        def g_start(j, buf, sem):
            pltpu.async_copy(hp_hbm.at[src_v.at[j]], buf, sem)

        def g_wait(j, buf, sem):
            pltpu.make_async_copy(hp_hbm.at[src_v.at[j]], buf, sem).wait()

        def s_start(j, buf, sem):
            pltpu.async_copy(buf, acc.at[dst_v.at[j]], sem, add=True)

        def s_wait(j, buf, sem):
            pltpu.make_async_copy(buf, acc.at[dst_v.at[j]], sem).wait()

        g_start(0, rows_a, gsem_a)

        def body(jj, carry):
            j0 = 2 * jj
            j1 = j0 + 1
            g_wait(j0, rows_a, gsem_a)
            g_start(j1, rows_b, gsem_b)
            s_start(j0, rows_a, ssem_a)
            s_wait(j0, rows_a, ssem_a)

            @pl.when(jj != j_steps // 2 - 1)
            def _():
                g_start(j0 + 2, rows_a, gsem_a)

            g_wait(j1, rows_b, gsem_b)
            s_start(j1, rows_b, ssem_b)
            s_wait(j1, rows_b, ssem_b)
            return carry

        lax.fori_loop(0, j_steps // 2, body, 0)
        plsc.subcore_barrier()
        pltpu.sync_copy(acc.at[pl.ds(sid * slab, slab)],
                        out_hbm.at[cid, pl.ds(sid * slab, slab)])

    return k(hp2, src_t, dst_t, zslab)


def _tc_prep(deg_p, x, w1, n):
    n_pad = deg_p.shape[1]

    def body(degp_ref, x_ref, w_ref, dinv_ref, hp_ref):
        deg = degp_ref[0] + degp_ref[1] + 1.0
        dinv = lax.rsqrt(deg)
        dinv_ref[...] = dinv
        r = dinv[:n] * jnp.dot(
            x_ref[...], w_ref[...], preferred_element_type=jnp.float32)
        hp_ref[:n, :] = r
        hp_ref[n:, :] = r

    return pl.pallas_call(
        body,
        out_shape=[
            jax.ShapeDtypeStruct((n_pad, 1), jnp.float32),
            jax.ShapeDtypeStruct((2 * n, w1.shape[1]), jnp.float32),
        ],
    )(deg_p, x, w1)


def _tc_mid(agg_p, hp, dinv, b, w, n, split=False):
    def body(aggp_ref, hp_ref, dinv_ref, b_ref, w_ref, out_ref):
        agg = aggp_ref[0, :n, :] + aggp_ref[1, :n, :]
        dinv = dinv_ref[:n]
        act = jnp.maximum(dinv * (agg + hp_ref[:n, :]) + b_ref[...], 0.0)
        r = dinv * jnp.dot(act, w_ref[...], preferred_element_type=jnp.float32)
        if split:
            fh = w.shape[1] // 2
            out_ref[:n, :] = r[:, :fh]
            out_ref[n:, :] = r[:, fh:]
        else:
            out_ref[:n, :] = r
            out_ref[n:, :] = r

    out_shape = (jax.ShapeDtypeStruct((2 * n, w.shape[1] // 2), jnp.float32)
                 if split else
                 jax.ShapeDtypeStruct((2 * n, w.shape[1]), jnp.float32))
    return pl.pallas_call(body, out_shape=out_shape)(agg_p, hp, dinv, b, w)


def _tc_final(agg_p, hp2, dinv, b4, wf1, bf1, wf2, bf2, n):
    def body(aggp_ref, hp_ref, dinv_ref, b4_ref, wf1_ref, bf1_ref,
             wf2_ref, bf2_ref, out_ref):
        agg = jnp.concatenate(
            [aggp_ref[0, :n, :], aggp_ref[1, :n, :]], axis=1)
        hp = jnp.concatenate([hp_ref[:n, :], hp_ref[n:, :]], axis=1)
        dinv = dinv_ref[:n]
        act = jnp.maximum(dinv * (agg + hp) + b4_ref[...], 0.0)
        z = jnp.maximum(jnp.dot(act, wf1_ref[...],
                                preferred_element_type=jnp.float32)
                        + bf1_ref[...], 0.0)
        out_ref[...] = jnp.tanh(
            jnp.dot(z, wf2_ref[...], preferred_element_type=jnp.float32)
            + bf2_ref[...])

    return pl.pallas_call(
        body,
        out_shape=jax.ShapeDtypeStruct((n, wf2.shape[1]), jnp.float32),
    )(agg_p, hp2, dinv, b4, wf1, bf1, wf2, bf2)


def kernel(x, edge_index, W1, b1, W2, b2, W3, b3, W4, b4, Wf1, bf1, Wf2, bf2):
    n = x.shape[0]
    e = edge_index.shape[1]

    slab = ((n + 1 + NS * 64 - 1) // (NS * 64)) * 64
    n_pad = slab * NS

    epg = NW * K
    e_pad = ((e + 2 * epg - 1) // (2 * epg)) * (2 * epg)
    src = edge_index[0]
    dst = edge_index[1]
    if e_pad != e:
        src = jnp.concatenate([src, jnp.zeros((e_pad - e,), jnp.int32)])
        dst = jnp.concatenate([dst, jnp.full((e_pad - e,), n, jnp.int32)])
    j_steps = e_pad // epg
    src_es = src + n * (jnp.arange(e_pad, dtype=jnp.int32)
                        >= (e_pad // 2)).astype(jnp.int32)
    src_t = src_es.reshape(NW, j_steps, K)
    dst_t = dst.reshape(NW, j_steps, K)

    zslab1 = jnp.zeros((slab,), jnp.float32)
    ones_k = jnp.ones((K,), jnp.float32)

    deg_p = _deg_partials(dst_t, zslab1, ones_k, n_pad, slab)
    dinv, hp = _tc_prep(deg_p.reshape(NC, n_pad, 1), x, W1, n)

    for b_l, w_next in ((b1, W2), (b2, W3)):
        f = hp.shape[1]
        agg_p = _segsum_partials(hp, src_t, dst_t,
                                 jnp.zeros((slab, f), jnp.float32),
                                 n_pad, slab)
        hp = _tc_mid(agg_p, hp, dinv, b_l.reshape(1, -1), w_next, n)

    f = hp.shape[1]
    agg_p = _segsum_partials(hp, src_t, dst_t,
                             jnp.zeros((slab, f), jnp.float32), n_pad, slab)
    hp2 = _tc_mid(agg_p, hp, dinv, b3.reshape(1, -1), W4, n, split=True)

    src2 = jnp.stack([src, src + n])
    fh = hp2.shape[1]
    agg_p = _segsum_featsplit(hp2, src2, dst,
                              jnp.zeros((slab, fh), jnp.float32),
                              n_pad, slab)
    return _tc_final(agg_p, hp2, dinv, b4.reshape(1, -1), Wf1,
                     bf1.reshape(1, -1), Wf2, bf2.reshape(1, -1), n)


# speedup vs baseline: 1.3128x; 1.3128x over previous
import functools

import jax
import jax.numpy as jnp
from jax import lax
from jax.experimental import pallas as pl
from jax.experimental.pallas import tpu as pltpu
from jax.experimental.pallas import tpu_sc as plsc

NC = 2
NS = 16
NW = NC * NS
K = 80
FW = 128


def _sc_mesh():
    return plsc.VectorSubcoreMesh(
        core_axis_name="c", subcore_axis_name="s",
        num_cores=NC, num_subcores=NS)


def _deg_partials(dst_t, zslab, ones_k, n_pad, slab):
    j_steps = dst_t.shape[1]

    @functools.partial(
        pl.kernel,
        out_type=jax.ShapeDtypeStruct((NC * n_pad,), jnp.float32),
        mesh=_sc_mesh(),
        scratch_types=[
            pltpu.VMEM((j_steps, K), jnp.int32),
            pltpu.VMEM((K,), jnp.float32),
            pltpu.VMEM_SHARED((n_pad,), jnp.float32),
            pltpu.SemaphoreType.DMA,
        ],
    )
    def k(dst_hbm, z_hbm, ones_hbm, out_hbm, idx_v, ones_v, acc, sem):
        cid = lax.axis_index("c")
        sid = lax.axis_index("s")
        gid = cid * NS + sid
        pltpu.sync_copy(z_hbm, acc.at[pl.ds(sid * slab, slab)])
        pltpu.sync_copy(ones_hbm, ones_v)
        pltpu.sync_copy(dst_hbm.at[gid], idx_v)
        plsc.subcore_barrier()

        def body(j, carry):
            pltpu.sync_copy(ones_v, acc.at[idx_v.at[j]], add=True)
            return carry

        lax.fori_loop(0, j_steps, body, 0)
        plsc.subcore_barrier()
        pltpu.sync_copy(acc.at[pl.ds(sid * slab, slab)],
                        out_hbm.at[pl.ds(cid * n_pad + sid * slab, slab)])

    return k(dst_t, zslab, ones_k)


def _segsum_partials(hp, src_t, dst_t, zslab, n_pad, slab):
    f = hp.shape[1]
    kk = {16: 1008, 32: 720, 64: 240, 128: 48}[f]
    src_t = src_t.reshape(NW, -1, kk)
    dst_t = dst_t.reshape(NW, -1, kk)
    j_steps = src_t.shape[1]

    assert j_steps % 2 == 0

    @functools.partial(
        pl.kernel,
        out_type=jax.ShapeDtypeStruct((NC, n_pad, f), jnp.float32),
        mesh=_sc_mesh(),
        scratch_types=[
            pltpu.VMEM_SHARED((n_pad, f), jnp.float32),
            pltpu.VMEM((j_steps, kk), jnp.int32),
            pltpu.VMEM((j_steps, kk), jnp.int32),
            pltpu.VMEM((kk, f), jnp.float32),
            pltpu.VMEM((kk, f), jnp.float32),
            pltpu.SemaphoreType.DMA,
            pltpu.SemaphoreType.DMA,
            pltpu.SemaphoreType.DMA,
            pltpu.SemaphoreType.DMA,
        ],
        compiler_params=pltpu.CompilerParams(use_tc_tiling_on_sc=False),
    )
    def k(hp_hbm, src_hbm, dst_hbm, z_hbm, out_hbm,
          acc, src_v, dst_v, rows_a, rows_b, gsem_a, gsem_b, ssem_a, ssem_b):
        cid = lax.axis_index("c")
        sid = lax.axis_index("s")
        gid = cid * NS + sid
        pltpu.sync_copy(z_hbm, acc.at[pl.ds(sid * slab, slab)])
        pltpu.sync_copy(src_hbm.at[gid], src_v)
        pltpu.sync_copy(dst_hbm.at[gid], dst_v)
        plsc.subcore_barrier()

        def g_start(j, buf, sem):
            pltpu.async_copy(hp_hbm.at[src_v.at[j]], buf, sem)

        def g_wait(j, buf, sem):
            pltpu.make_async_copy(hp_hbm.at[src_v.at[j]], buf, sem).wait()

        def s_start(j, buf, sem):
            pltpu.async_copy(buf, acc.at[dst_v.at[j]], sem, add=True)

        def s_wait(j, buf, sem):
            pltpu.make_async_copy(buf, acc.at[dst_v.at[j]], sem).wait()

        g_start(0, rows_a, gsem_a)

        def body(jj, carry):
            j0 = 2 * jj
            j1 = j0 + 1
            g_wait(j0, rows_a, gsem_a)
            g_start(j1, rows_b, gsem_b)
            s_start(j0, rows_a, ssem_a)
            s_wait(j0, rows_a, ssem_a)

            @pl.when(jj != j_steps // 2 - 1)
            def _():
                g_start(j0 + 2, rows_a, gsem_a)

            g_wait(j1, rows_b, gsem_b)
            s_start(j1, rows_b, ssem_b)
            s_wait(j1, rows_b, ssem_b)
            return carry

        lax.fori_loop(0, j_steps // 2, body, 0)
        plsc.subcore_barrier()
        pltpu.sync_copy(acc.at[pl.ds(sid * slab, slab)],
                        out_hbm.at[cid, pl.ds(sid * slab, slab)])

    return k(hp, src_t, dst_t, zslab)


def _segsum_featsplit(hp2, src2, dst, zslab, n_pad, slab):
    fh = hp2.shape[1]
    kk = 240
    src_t = src2.reshape(NC * NS, -1, kk)
    dst_t = dst.reshape(NS, -1, kk)
    j_steps = src_t.shape[1]
    assert j_steps % 2 == 0 and j_steps == dst_t.shape[1]

    @functools.partial(
        pl.kernel,
        out_type=jax.ShapeDtypeStruct((NC, n_pad, fh), jnp.float32),
        mesh=_sc_mesh(),
        scratch_types=[
            pltpu.VMEM_SHARED((n_pad, fh), jnp.float32),
            pltpu.VMEM((j_steps, kk), jnp.int32),
            pltpu.VMEM((j_steps, kk), jnp.int32),
            pltpu.VMEM((kk, fh), jnp.float32),
            pltpu.VMEM((kk, fh), jnp.float32),
            pltpu.SemaphoreType.DMA,
            pltpu.SemaphoreType.DMA,
            pltpu.SemaphoreType.DMA,
            pltpu.SemaphoreType.DMA,
        ],
        compiler_params=pltpu.CompilerParams(use_tc_tiling_on_sc=False),
    )
    def k(hp_hbm, src_hbm, dst_hbm, z_hbm, out_hbm,
          acc, src_v, dst_v, rows_a, rows_b,
          gsem_a, gsem_b, ssem_a, ssem_b):
        cid = lax.axis_index("c")
        sid = lax.axis_index("s")
        pltpu.sync_copy(z_hbm, acc.at[pl.ds(sid * slab, slab)])
        pltpu.sync_copy(src_hbm.at[cid * NS + sid], src_v)
        pltpu.sync_copy(dst_hbm.at[sid], dst_v)
        plsc.subcore_barrier()

        def g_start(j, buf, sem):
            pltpu.async_copy(hp_hbm.at[src_v.at[j]], buf, sem)

        def g_wait(j, buf, sem):
            pltpu.make_async_copy(hp_hbm.at[src_v.at[j]], buf, sem).wait()

        def s_start(j, buf, sem):
            pltpu.async_copy(buf, acc.at[dst_v.at[j]], sem, add=True)

        def s_wait(j, buf, sem):
            pltpu.make_async_copy(buf, acc.at[dst_v.at[j]], sem).wait()

        g_start(0, rows_a, gsem_a)

        def body(jj, carry):
            j0 = 2 * jj
            j1 = j0 + 1
            g_wait(j0, rows_a, gsem_a)
            g_start(j1, rows_b, gsem_b)
            s_start(j0, rows_a, ssem_a)
            s_wait(j0, rows_a, ssem_a)

            @pl.when(jj != j_steps // 2 - 1)
            def _():
                g_start(j0 + 2, rows_a, gsem_a)

            g_wait(j1, rows_b, gsem_b)
            s_start(j1, rows_b, ssem_b)
            s_wait(j1, rows_b, ssem_b)
            return carry

        lax.fori_loop(0, j_steps // 2, body, 0)
        plsc.subcore_barrier()
        pltpu.sync_copy(acc.at[pl.ds(sid * slab, slab)],
                        out_hbm.at[cid, pl.ds(sid * slab, slab)])

    return k(hp2, src_t, dst_t, zslab)


def _tc_prep(deg_p, x, w1, n):
    n_pad = deg_p.shape[1]

    def body(degp_ref, x_ref, w_ref, dinv_ref, hp_ref):
        deg = degp_ref[0] + degp_ref[1] + 1.0
        dinv = lax.rsqrt(deg)
        dinv_ref[...] = dinv
        hp_ref[...] = dinv[:n] * jnp.dot(
            x_ref[...], w_ref[...], preferred_element_type=jnp.float32)

    return pl.pallas_call(
        body,
        out_shape=[
            jax.ShapeDtypeStruct((n_pad, 1), jnp.float32),
            jax.ShapeDtypeStruct((n, w1.shape[1]), jnp.float32),
        ],
    )(deg_p, x, w1)


def _tc_combine(agg_p, hp, dinv, b, n):
    def body(aggp_ref, hp_ref, dinv_ref, b_ref, out_ref):
        agg = aggp_ref[0, :n, :] + aggp_ref[1, :n, :]
        dinv = dinv_ref[:n]
        out_ref[...] = dinv * jnp.maximum(
            dinv * (agg + hp_ref[...]) + b_ref[...], 0.0)

    return pl.pallas_call(
        body,
        out_shape=jax.ShapeDtypeStruct(hp.shape, jnp.float32),
    )(agg_p, hp, dinv, b)


def _tc_mid(agg_p, hp, dinv, b, w, n):
    def body(aggp_ref, hp_ref, dinv_ref, b_ref, w_ref, out_ref):
        agg = aggp_ref[0, :n, :] + aggp_ref[1, :n, :]
        dinv = dinv_ref[:n]
        m = dinv * (agg + hp_ref[...])
        act = jnp.maximum(
            jnp.dot(m, w_ref[...], preferred_element_type=jnp.float32)
            + b_ref[...], 0.0)
        out_ref[...] = dinv * act

    return pl.pallas_call(
        body,
        out_shape=jax.ShapeDtypeStruct((n, w.shape[1]), jnp.float32),
    )(agg_p, hp, dinv, b, w)


def _tc_final(agg_p, hp, dinv, b4, w4, wf1, bf1, wf2, bf2, n):
    def body(aggp_ref, hp_ref, dinv_ref, b4_ref, w4_ref, wf1_ref, bf1_ref,
             wf2_ref, bf2_ref, out_ref):
        agg = aggp_ref[0, :n, :] + aggp_ref[1, :n, :]
        dinv = dinv_ref[:n]
        m = dinv * (agg + hp_ref[...])
        act = jnp.maximum(
            jnp.dot(m, w4_ref[...], preferred_element_type=jnp.float32)
            + b4_ref[...], 0.0)
        z = jnp.maximum(jnp.dot(act, wf1_ref[...],
                                preferred_element_type=jnp.float32)
                        + bf1_ref[...], 0.0)
        out_ref[...] = jnp.tanh(
            jnp.dot(z, wf2_ref[...], preferred_element_type=jnp.float32)
            + bf2_ref[...])

    return pl.pallas_call(
        body,
        out_shape=jax.ShapeDtypeStruct((n, wf2.shape[1]), jnp.float32),
    )(agg_p, hp, dinv, b4, w4, wf1, bf1, wf2, bf2)


def kernel(x, edge_index, W1, b1, W2, b2, W3, b3, W4, b4, Wf1, bf1, Wf2, bf2):
    n = x.shape[0]
    e = edge_index.shape[1]

    slab = ((n + 1 + NS * 64 - 1) // (NS * 64)) * 64
    n_pad = slab * NS

    epg = NW * K
    e_pad = ((e + 2 * epg - 1) // (2 * epg)) * (2 * epg)
    src = edge_index[0]
    dst = edge_index[1]
    if e_pad != e:
        src = jnp.concatenate([src, jnp.zeros((e_pad - e,), jnp.int32)])
        dst = jnp.concatenate([dst, jnp.full((e_pad - e,), n, jnp.int32)])
    j_steps = e_pad // epg
    src_t = src.reshape(NW, j_steps, K)
    dst_t = dst.reshape(NW, j_steps, K)

    zslab1 = jnp.zeros((slab,), jnp.float32)
    ones_k = jnp.ones((K,), jnp.float32)

    def segsum(hp):
        f = hp.shape[1]
        return _segsum_partials(hp, src_t, dst_t,
                                jnp.zeros((slab, f), jnp.float32),
                                n_pad, slab)

    deg_p = _deg_partials(dst_t, zslab1, ones_k, n_pad, slab)
    dinv, hp = _tc_prep(deg_p.reshape(NC, n_pad, 1), x, W1, n)

    hp = _tc_combine(segsum(hp), hp, dinv, b1.reshape(1, -1), n)
    hp = _tc_mid(segsum(hp), hp, dinv, b2.reshape(1, -1), W2, n)
    hp = _tc_mid(segsum(hp), hp, dinv, b3.reshape(1, -1), W3, n)
    return _tc_final(segsum(hp), hp, dinv, b4.reshape(1, -1), W4, Wf1,
                     bf1.reshape(1, -1), Wf2, bf2.reshape(1, -1), n)


# speedup vs baseline: 1.4035x; 1.0691x over previous
import functools

import jax
import jax.numpy as jnp
from jax import lax
from jax.experimental import pallas as pl
from jax.experimental.pallas import tpu as pltpu
from jax.experimental.pallas import tpu_sc as plsc

NC = 2
NS = 16
NW = NC * NS
K = 80
FW = 128


def _sc_mesh():
    return plsc.VectorSubcoreMesh(
        core_axis_name="c", subcore_axis_name="s",
        num_cores=NC, num_subcores=NS)


def _deg_partials(dst_t, zslab, ones_k, n_pad, slab):
    j_steps = dst_t.shape[1]

    @functools.partial(
        pl.kernel,
        out_type=jax.ShapeDtypeStruct((NC * n_pad,), jnp.float32),
        mesh=_sc_mesh(),
        scratch_types=[
            pltpu.VMEM((j_steps, K), jnp.int32),
            pltpu.VMEM((K,), jnp.float32),
            pltpu.VMEM_SHARED((n_pad,), jnp.float32),
            pltpu.SemaphoreType.DMA,
        ],
    )
    def k(dst_hbm, z_hbm, ones_hbm, out_hbm, idx_v, ones_v, acc, sem):
        cid = lax.axis_index("c")
        sid = lax.axis_index("s")
        gid = cid * NS + sid
        pltpu.sync_copy(z_hbm, acc.at[pl.ds(sid * slab, slab)])
        pltpu.sync_copy(ones_hbm, ones_v)
        pltpu.sync_copy(dst_hbm.at[gid], idx_v)
        plsc.subcore_barrier()

        def body(j, carry):
            pltpu.sync_copy(ones_v, acc.at[idx_v.at[j]], add=True)
            return carry

        lax.fori_loop(0, j_steps, body, 0)
        plsc.subcore_barrier()
        pltpu.sync_copy(acc.at[pl.ds(sid * slab, slab)],
                        out_hbm.at[pl.ds(cid * n_pad + sid * slab, slab)])

    return k(dst_t, zslab, ones_k)


def _segsum_partials(hp, src_t, dst_t, zslab, n_pad, slab):
    f = hp.shape[1]
    kk = {16: 1008, 32: 720, 64: 240, 128: 48}[f]
    src_t = src_t.reshape(NW, -1, kk)
    dst_t = dst_t.reshape(NW, -1, kk)
    j_steps = src_t.shape[1]

    assert j_steps % 2 == 0

    @functools.partial(
        pl.kernel,
        out_type=jax.ShapeDtypeStruct((NC, n_pad, f), jnp.float32),
        mesh=_sc_mesh(),
        scratch_types=[
            pltpu.VMEM_SHARED((n_pad, f), jnp.float32),
            pltpu.VMEM((j_steps, kk), jnp.int32),
            pltpu.VMEM((j_steps, kk), jnp.int32),
            pltpu.VMEM((kk, f), jnp.float32),
            pltpu.VMEM((kk, f), jnp.float32),
            pltpu.SemaphoreType.DMA,
            pltpu.SemaphoreType.DMA,
            pltpu.SemaphoreType.DMA,
            pltpu.SemaphoreType.DMA,
        ],
        compiler_params=pltpu.CompilerParams(use_tc_tiling_on_sc=False),
    )
    def k(hp_hbm, src_hbm, dst_hbm, z_hbm, out_hbm,
          acc, src_v, dst_v, rows_a, rows_b, gsem_a, gsem_b, ssem_a, ssem_b):
        cid = lax.axis_index("c")
        sid = lax.axis_index("s")
        gid = cid * NS + sid
        pltpu.sync_copy(z_hbm, acc.at[pl.ds(sid * slab, slab)])
        pltpu.sync_copy(src_hbm.at[gid], src_v)
        pltpu.sync_copy(dst_hbm.at[gid], dst_v)
        plsc.subcore_barrier()

        def g_start(j, buf, sem):
            pltpu.async_copy(hp_hbm.at[src_v.at[j]], buf, sem)

        def g_wait(j, buf, sem):
            pltpu.make_async_copy(hp_hbm.at[src_v.at[j]], buf, sem).wait()

        def s_start(j, buf, sem):
            pltpu.async_copy(buf, acc.at[dst_v.at[j]], sem, add=True)

        def s_wait(j, buf, sem):
            pltpu.make_async_copy(buf, acc.at[dst_v.at[j]], sem).wait()

        g_start(0, rows_a, gsem_a)

        def body(jj, carry):
            j0 = 2 * jj
            j1 = j0 + 1
            g_wait(j0, rows_a, gsem_a)
            g_start(j1, rows_b, gsem_b)
            s_start(j0, rows_a, ssem_a)
            s_wait(j0, rows_a, ssem_a)

            @pl.when(jj != j_steps // 2 - 1)
            def _():
                g_start(j0 + 2, rows_a, gsem_a)

            g_wait(j1, rows_b, gsem_b)
            s_start(j1, rows_b, ssem_b)
            s_wait(j1, rows_b, ssem_b)
            return carry

        lax.fori_loop(0, j_steps // 2, body, 0)
        plsc.subcore_barrier()
        pltpu.sync_copy(acc.at[pl.ds(sid * slab, slab)],
                        out_hbm.at[cid, pl.ds(sid * slab, slab)])

    return k(hp, src_t, dst_t, zslab)


def _segsum_featsplit(hp2, src2, dst, zslab, n_pad, slab):
    fh = hp2.shape[1]
    kk = {8: 672, 16: 672, 32: 480}[fh]
    src_t = src2.reshape(NC * NS, -1, kk)
    dst_t = dst.reshape(NS, -1, kk)
    j_steps = src_t.shape[1]
    assert j_steps % 2 == 0 and j_steps == dst_t.shape[1]

    @functools.partial(
        pl.kernel,
        out_type=jax.ShapeDtypeStruct((NC, n_pad, fh), jnp.float32),
        mesh=_sc_mesh(),
        scratch_types=[
            pltpu.VMEM_SHARED((n_pad, fh), jnp.float32),
            pltpu.VMEM((j_steps, kk), jnp.int32),
            pltpu.VMEM((j_steps, kk), jnp.int32),
            pltpu.VMEM((kk, fh), jnp.float32),
            pltpu.VMEM((kk, fh), jnp.float32),
            pltpu.SemaphoreType.DMA,
            pltpu.SemaphoreType.DMA,
            pltpu.SemaphoreType.DMA,
            pltpu.SemaphoreType.DMA,
        ],
        compiler_params=pltpu.CompilerParams(use_tc_tiling_on_sc=False),
    )
    def k(hp_hbm, src_hbm, dst_hbm, z_hbm, out_hbm,
          acc, src_v, dst_v, rows_a, rows_b,
          gsem_a, gsem_b, ssem_a, ssem_b):
        cid = lax.axis_index("c")
        sid = lax.axis_index("s")
        pltpu.sync_copy(z_hbm, acc.at[pl.ds(sid * slab, slab)])
        pltpu.sync_copy(src_hbm.at[cid * NS + sid], src_v)
        pltpu.sync_copy(dst_hbm.at[sid], dst_v)
        plsc.subcore_barrier()

        def g_start(j, buf, sem):
            pltpu.async_copy(hp_hbm.at[src_v.at[j]], buf, sem)

        def g_wait(j, buf, sem):
            pltpu.make_async_copy(hp_hbm.at[src_v.at[j]], buf, sem).wait()

        def s_start(j, buf, sem):
            pltpu.async_copy(buf, acc.at[dst_v.at[j]], sem, add=True)

        def s_wait(j, buf, sem):
            pltpu.make_async_copy(buf, acc.at[dst_v.at[j]], sem).wait()

        g_start(0, rows_a, gsem_a)

        def body(jj, carry):
            j0 = 2 * jj
            j1 = j0 + 1
            g_wait(j0, rows_a, gsem_a)
            g_start(j1, rows_b, gsem_b)
            s_start(j0, rows_a, ssem_a)
            s_wait(j0, rows_a, ssem_a)

            @pl.when(jj != j_steps // 2 - 1)
            def _():
                g_start(j0 + 2, rows_a, gsem_a)

            g_wait(j1, rows_b, gsem_b)
            s_start(j1, rows_b, ssem_b)
            s_wait(j1, rows_b, ssem_b)
            return carry

        lax.fori_loop(0, j_steps // 2, body, 0)
        plsc.subcore_barrier()
        pltpu.sync_copy(acc.at[pl.ds(sid * slab, slab)],
                        out_hbm.at[cid, pl.ds(sid * slab, slab)])

    return k(hp2, src_t, dst_t, zslab)


def _tc_prep(deg_p, x, w1, n):
    n_pad = deg_p.shape[1]

    def body(degp_ref, x_ref, w_ref, dinv_ref, hp_ref):
        deg = degp_ref[0] + degp_ref[1] + 1.0
        dinv = lax.rsqrt(deg)
        dinv_ref[...] = dinv
        hp_ref[...] = dinv[:n] * jnp.dot(
            x_ref[...], w_ref[...], preferred_element_type=jnp.float32)

    return pl.pallas_call(
        body,
        out_shape=[
            jax.ShapeDtypeStruct((n_pad, 1), jnp.float32),
            jax.ShapeDtypeStruct((n, w1.shape[1]), jnp.float32),
        ],
    )(deg_p, x, w1)


def _tc_combine(agg_p, hp, dinv, b, n):
    def body(aggp_ref, hp_ref, dinv_ref, b_ref, out_ref):
        agg = aggp_ref[0, :n, :] + aggp_ref[1, :n, :]
        dinv = dinv_ref[:n]
        out_ref[...] = dinv * jnp.maximum(
            dinv * (agg + hp_ref[...]) + b_ref[...], 0.0)

    return pl.pallas_call(
        body,
        out_shape=jax.ShapeDtypeStruct(hp.shape, jnp.float32),
    )(agg_p, hp, dinv, b)


def _tc_mid(agg_p, hp, dinv, b, w, n, in_split=False, out_split=False):
    def body(aggp_ref, hp_ref, dinv_ref, b_ref, w_ref, out_ref):
        if in_split:
            agg = jnp.concatenate(
                [aggp_ref[0, :n, :], aggp_ref[1, :n, :]], axis=1)
            hpv = jnp.concatenate([hp_ref[:n, :], hp_ref[n:, :]], axis=1)
        else:
            agg = aggp_ref[0, :n, :] + aggp_ref[1, :n, :]
            hpv = hp_ref[...]
        dinv = dinv_ref[:n]
        m = dinv * (agg + hpv)
        act = jnp.maximum(
            jnp.dot(m, w_ref[...], preferred_element_type=jnp.float32)
            + b_ref[...], 0.0)
        r = dinv * act
        if out_split:
            fh = w.shape[1] // 2
            out_ref[:n, :] = r[:, :fh]
            out_ref[n:, :] = r[:, fh:]
        else:
            out_ref[...] = r

    out_shape = (jax.ShapeDtypeStruct((2 * n, w.shape[1] // 2), jnp.float32)
                 if out_split else
                 jax.ShapeDtypeStruct((n, w.shape[1]), jnp.float32))
    return pl.pallas_call(body, out_shape=out_shape)(agg_p, hp, dinv, b, w)


def _tc_final(agg_p, hp, dinv, b4, w4, wf1, bf1, wf2, bf2, n):
    def body(aggp_ref, hp_ref, dinv_ref, b4_ref, w4_ref, wf1_ref, bf1_ref,
             wf2_ref, bf2_ref, out_ref):
        agg = jnp.concatenate(
            [aggp_ref[0, :n, :], aggp_ref[1, :n, :]], axis=1)
        hpv = jnp.concatenate([hp_ref[:n, :], hp_ref[n:, :]], axis=1)
        dinv = dinv_ref[:n]
        m = dinv * (agg + hpv)
        act = jnp.maximum(
            jnp.dot(m, w4_ref[...], preferred_element_type=jnp.float32)
            + b4_ref[...], 0.0)
        z = jnp.maximum(jnp.dot(act, wf1_ref[...],
                                preferred_element_type=jnp.float32)
                        + bf1_ref[...], 0.0)
        out_ref[...] = jnp.tanh(
            jnp.dot(z, wf2_ref[...], preferred_element_type=jnp.float32)
            + bf2_ref[...])

    return pl.pallas_call(
        body,
        out_shape=jax.ShapeDtypeStruct((n, wf2.shape[1]), jnp.float32),
    )(agg_p, hp, dinv, b4, w4, wf1, bf1, wf2, bf2)


def kernel(x, edge_index, W1, b1, W2, b2, W3, b3, W4, b4, Wf1, bf1, Wf2, bf2):
    n = x.shape[0]
    e = edge_index.shape[1]

    slab = ((n + 1 + NS * 64 - 1) // (NS * 64)) * 64
    n_pad = slab * NS

    epg = NW * K
    e_pad = ((e + 2 * epg - 1) // (2 * epg)) * (2 * epg)
    src = edge_index[0]
    dst = edge_index[1]
    if e_pad != e:
        src = jnp.concatenate([src, jnp.zeros((e_pad - e,), jnp.int32)])
        dst = jnp.concatenate([dst, jnp.full((e_pad - e,), n, jnp.int32)])
    j_steps = e_pad // epg
    src_t = src.reshape(NW, j_steps, K)
    dst_t = dst.reshape(NW, j_steps, K)

    zslab1 = jnp.zeros((slab,), jnp.float32)
    ones_k = jnp.ones((K,), jnp.float32)

    def segsum(hp):
        f = hp.shape[1]
        return _segsum_partials(hp, src_t, dst_t,
                                jnp.zeros((slab, f), jnp.float32),
                                n_pad, slab)

    src2 = jnp.stack([src, src + n])

    def segsum_fs(hp2):
        fh = hp2.shape[1]
        return _segsum_featsplit(hp2, src2, dst,
                                 jnp.zeros((slab, fh), jnp.float32),
                                 n_pad, slab)

    deg_p = _deg_partials(dst_t, zslab1, ones_k, n_pad, slab)
    dinv, hp = _tc_prep(deg_p.reshape(NC, n_pad, 1), x, W1, n)

    hp = _tc_combine(segsum(hp), hp, dinv, b1.reshape(1, -1), n)
    hp = _tc_mid(segsum(hp), hp, dinv, b2.reshape(1, -1), W2, n,
                 out_split=True)
    hp = _tc_mid(segsum_fs(hp), hp, dinv, b3.reshape(1, -1), W3, n,
                 in_split=True, out_split=True)
    return _tc_final(segsum_fs(hp), hp, dinv, b4.reshape(1, -1), W4, Wf1,
                     bf1.reshape(1, -1), Wf2, bf2.reshape(1, -1), n)
